# baseline (speedup 1.0000x reference)
import jax
import jax.numpy as jnp
import numpy as np
from jax.experimental import pallas as pl
from jax.experimental.pallas import tpu as pltpu

_N, _C, _H, _W = 4, 96, 224, 224
_HW = _H * _W
_RATIO = 2
_MAXN = _HW * _RATIO
_CB = 8
_NCB = _C // _CB


def _rotl(x, d):
    return ((x << np.uint32(d)) | (x >> np.uint32(32 - d))).astype(np.uint32)


def _threefry2x32(k0, k1, x0, x1):
    x0 = x0.astype(np.uint32).copy()
    x1 = x1.astype(np.uint32).copy()
    ks2 = np.uint32(k0 ^ k1 ^ np.uint32(0x1BD11BDA))
    rot = [(13, 15, 26, 6), (17, 29, 16, 24)]
    x0 = (x0 + k0).astype(np.uint32)
    x1 = (x1 + k1).astype(np.uint32)
    ks = [k0, k1, ks2]
    for i in range(5):
        for r in rot[i % 2]:
            x0 = (x0 + x1).astype(np.uint32)
            x1 = _rotl(x1, r) ^ x0
        x0 = (x0 + ks[(i + 1) % 3]).astype(np.uint32)
        x1 = (x1 + ks[(i + 2) % 3] + np.uint32(i + 1)).astype(np.uint32)
    return x0, x1


def _np_fold_in(key, data):
    o0, o1 = _threefry2x32(key[0], key[1], np.array([0], np.uint32),
                           np.array([data], np.uint32))
    return np.array([o0[0], o1[0]], np.uint32)


def _np_random_bits(key, n):
    b1, b2 = _threefry2x32(key[0], key[1], np.zeros(n, np.uint32),
                           np.arange(n, dtype=np.uint32))
    return b1 ^ b2


def _np_split(key):
    b1, b2 = _threefry2x32(key[0], key[1], np.zeros(2, np.uint32),
                           np.array([0, 1], np.uint32))
    return (np.array([b1[0], b2[0]], np.uint32),
            np.array([b1[1], b2[1]], np.uint32))


def _np_randint(key, n, maxval):
    k1, k2 = _np_split(key)
    y = _np_random_bits(k1, n)
    z = _np_random_bits(k2, n)
    s = np.uint32(maxval)
    mult = ((np.uint32(65536) % s) ** 2) % s
    return (((y % s) * mult + (z % s)) % s).astype(np.int64)


def _first_hit_table() -> np.ndarray:
    base = np.array([0, 1234], np.uint32)
    rows = []
    js_rev = np.arange(_MAXN, dtype=np.int32)[::-1]
    for i in range(_N):
        xs = _np_randint(_np_fold_in(base, 2 * i), _MAXN, _W)
        ys = _np_randint(_np_fold_in(base, 2 * i + 1), _MAXN, _H)
        pos = ys * _W + xs
        fh = np.full(_HW, _MAXN, np.int32)
        fh[pos[::-1]] = js_rev
        rows.append(fh)
    return np.stack(rows)


_FH = _first_hit_table()


_K = 4
_STEPS = _N * _NCB


def _body(pred_hbm, target_hbm, fh_ref, out_ref,
          pbuf, tbuf, psem, tsem, sacc, tacc, acc_ref):
    s = pl.program_id(0)

    def _start(step):
        i2 = step // _NCB
        cb2 = step % _NCB
        k = step % _K
        for j in range(_CB // 2):
            pltpu.make_async_copy(
                pred_hbm.at[i2, pl.ds(cb2 * _CB + 2 * j, 2), :],
                pbuf.at[k, pl.ds(2 * j, 2), :], psem.at[k]).start()
            pltpu.make_async_copy(
                target_hbm.at[i2, pl.ds(cb2 * _CB + 2 * j, 2), :],
                tbuf.at[k, pl.ds(2 * j, 2), :], tsem.at[k]).start()

    @pl.when(s == 0)
    def _():
        for k in range(_K):
            _start(k)

    k = s % _K
    i = s // _NCB
    cb = s % _NCB
    for j in range(_CB // 2):
        pltpu.make_async_copy(
            pred_hbm.at[i, pl.ds(cb * _CB + 2 * j, 2), :],
            pbuf.at[k, pl.ds(2 * j, 2), :], psem.at[k]).wait()
        pltpu.make_async_copy(
            target_hbm.at[i, pl.ds(cb * _CB + 2 * j, 2), :],
            tbuf.at[k, pl.ds(2 * j, 2), :], tsem.at[k]).wait()

    x = pbuf[k]
    z = tbuf[k]
    bce = jnp.maximum(x, 0.0) - x * z + jnp.log1p(jnp.exp(-jnp.abs(x)))

    @pl.when(cb == 0)
    def _():
        sacc[...] = bce
        tacc[...] = z

    @pl.when(cb != 0)
    def _():
        sacc[...] += bce
        tacc[...] = jnp.maximum(tacc[...], z)

    @pl.when(s + _K < _STEPS)
    def _():
        _start(s + _K)

    @pl.when(cb == _NCB - 1)
    def _():
        tm = jnp.max(tacc[...], axis=0)
        num = jnp.sum(tm).astype(jnp.int32) * _RATIO
        srow = jnp.sum(sacc[...], axis=0)
        m = (tm > 0.0).astype(jnp.float32)
        w = jnp.maximum(m, (fh_ref[0, 0] < num).astype(jnp.float32))
        n_part = jnp.sum(w * srow)
        d_part = jnp.sum(w)

        @pl.when(i == 0)
        def _():
            acc_ref[0] = n_part
            acc_ref[1] = d_part

        @pl.when(i != 0)
        def _():
            acc_ref[0] += n_part
            acc_ref[1] += d_part

        @pl.when(i == _N - 1)
        def _():
            out_ref[0, 0] = acc_ref[0] / acc_ref[1]


@jax.jit
def _run(pred3, target3, fh):
    loss = pl.pallas_call(
        _body,
        grid=(_STEPS,),
        in_specs=[
            pl.BlockSpec(memory_space=pl.ANY),
            pl.BlockSpec(memory_space=pl.ANY),
            pl.BlockSpec((1, 1, _HW), lambda s: (s // _NCB, 0, 0)),
        ],
        out_specs=pl.BlockSpec((1, 1), lambda s: (0, 0),
                               memory_space=pltpu.SMEM),
        out_shape=jax.ShapeDtypeStruct((1, 1), jnp.float32),
        scratch_shapes=[
            pltpu.VMEM((_K, _CB, _HW), jnp.float32),
            pltpu.VMEM((_K, _CB, _HW), jnp.float32),
            pltpu.SemaphoreType.DMA((_K,)),
            pltpu.SemaphoreType.DMA((_K,)),
            pltpu.VMEM((_CB, _HW), jnp.float32),
            pltpu.VMEM((_CB, _HW), jnp.float32),
            pltpu.SMEM((2,), jnp.float32),
        ],
    )(pred3, target3, fh)
    return loss[0, 0]


def kernel(pred, target):
    pred3 = pred.reshape(_N, _C, _HW)
    target3 = target.reshape(_N, _C, _HW)
    return _run(pred3, target3, jnp.asarray(_FH).reshape(_N, 1, _HW))


# speedup vs baseline: 1.0510x; 1.0510x over previous
import jax
import jax.numpy as jnp
import numpy as np
from jax.experimental import pallas as pl
from jax.experimental.pallas import tpu as pltpu

_N, _C, _H, _W = 4, 96, 224, 224
_HW = _H * _W
_RATIO = 2
_MAXN = _HW * _RATIO
_CB = 8
_NCB = _C // _CB


def _rotl(x, d):
    return ((x << np.uint32(d)) | (x >> np.uint32(32 - d))).astype(np.uint32)


def _threefry2x32(k0, k1, x0, x1):
    x0 = x0.astype(np.uint32).copy()
    x1 = x1.astype(np.uint32).copy()
    ks2 = np.uint32(k0 ^ k1 ^ np.uint32(0x1BD11BDA))
    rot = [(13, 15, 26, 6), (17, 29, 16, 24)]
    x0 = (x0 + k0).astype(np.uint32)
    x1 = (x1 + k1).astype(np.uint32)
    ks = [k0, k1, ks2]
    for i in range(5):
        for r in rot[i % 2]:
            x0 = (x0 + x1).astype(np.uint32)
            x1 = _rotl(x1, r) ^ x0
        x0 = (x0 + ks[(i + 1) % 3]).astype(np.uint32)
        x1 = (x1 + ks[(i + 2) % 3] + np.uint32(i + 1)).astype(np.uint32)
    return x0, x1


def _np_fold_in(key, data):
    o0, o1 = _threefry2x32(key[0], key[1], np.array([0], np.uint32),
                           np.array([data], np.uint32))
    return np.array([o0[0], o1[0]], np.uint32)


def _np_random_bits(key, n):
    b1, b2 = _threefry2x32(key[0], key[1], np.zeros(n, np.uint32),
                           np.arange(n, dtype=np.uint32))
    return b1 ^ b2


def _np_split(key):
    b1, b2 = _threefry2x32(key[0], key[1], np.zeros(2, np.uint32),
                           np.array([0, 1], np.uint32))
    return (np.array([b1[0], b2[0]], np.uint32),
            np.array([b1[1], b2[1]], np.uint32))


def _np_randint(key, n, maxval):
    k1, k2 = _np_split(key)
    y = _np_random_bits(k1, n)
    z = _np_random_bits(k2, n)
    s = np.uint32(maxval)
    mult = ((np.uint32(65536) % s) ** 2) % s
    return (((y % s) * mult + (z % s)) % s).astype(np.int64)


def _first_hit_table() -> np.ndarray:
    base = np.array([0, 1234], np.uint32)
    rows = []
    js_rev = np.arange(_MAXN, dtype=np.int32)[::-1]
    for i in range(_N):
        xs = _np_randint(_np_fold_in(base, 2 * i), _MAXN, _W)
        ys = _np_randint(_np_fold_in(base, 2 * i + 1), _MAXN, _H)
        pos = ys * _W + xs
        fh = np.full(_HW, _MAXN, np.int32)
        fh[pos[::-1]] = js_rev
        rows.append(fh)
    return np.stack(rows)


_FH = _first_hit_table()


_TILE = 3584


def _dense_body(pred_ref, target_ref, s8_ref, t8_ref):
    def chunk(c, carry):
        sl = pl.ds(c * 128, 128)
        x = pred_ref[0, :, sl]
        z = target_ref[0, :, sl]
        bce = jnp.maximum(x, 0.0) - x * z + jnp.log1p(jnp.exp(-jnp.abs(x)))
        s = bce[0:8]
        zm = z[0:8]
        for r in range(8, _C, 8):
            s = s + bce[r:r + 8]
            zm = jnp.maximum(zm, z[r:r + 8])
        s8_ref[0, :, sl] = s
        t8_ref[0, :, sl] = zm
        return carry

    jax.lax.fori_loop(0, _TILE // 128, chunk, 0)


def _combine_body(s8_ref, t8_ref, fh_ref, out_ref, acc_ref):
    i = pl.program_id(0)
    tm = jnp.max(t8_ref[0], axis=0)
    num = jnp.sum(tm).astype(jnp.int32) * _RATIO
    s = jnp.sum(s8_ref[0], axis=0)
    m = (tm > 0.0).astype(jnp.float32)
    w = jnp.maximum(m, (fh_ref[0, 0] < num).astype(jnp.float32))
    n_part = jnp.sum(w * s)
    d_part = jnp.sum(w)

    @pl.when(i == 0)
    def _():
        acc_ref[0] = n_part
        acc_ref[1] = d_part

    @pl.when(i != 0)
    def _():
        acc_ref[0] += n_part
        acc_ref[1] += d_part

    @pl.when(i == _N - 1)
    def _():
        out_ref[0, 0] = acc_ref[0] / acc_ref[1]


@jax.jit
def _run(pred3, target3, fh):
    s8, t8 = pl.pallas_call(
        _dense_body,
        grid=(_N, _HW // _TILE),
        in_specs=[
            pl.BlockSpec((1, _C, _TILE), lambda i, t: (i, 0, t)),
            pl.BlockSpec((1, _C, _TILE), lambda i, t: (i, 0, t)),
        ],
        out_specs=[
            pl.BlockSpec((1, _CB, _TILE), lambda i, t: (i, 0, t)),
            pl.BlockSpec((1, _CB, _TILE), lambda i, t: (i, 0, t)),
        ],
        out_shape=[
            jax.ShapeDtypeStruct((_N, _CB, _HW), jnp.float32),
            jax.ShapeDtypeStruct((_N, _CB, _HW), jnp.float32),
        ],
    )(pred3, target3)

    loss = pl.pallas_call(
        _combine_body,
        grid=(_N,),
        in_specs=[
            pl.BlockSpec((1, _CB, _HW), lambda i: (i, 0, 0)),
            pl.BlockSpec((1, _CB, _HW), lambda i: (i, 0, 0)),
            pl.BlockSpec((1, 1, _HW), lambda i: (i, 0, 0)),
        ],
        out_specs=pl.BlockSpec((1, 1), lambda i: (0, 0),
                               memory_space=pltpu.SMEM),
        out_shape=jax.ShapeDtypeStruct((1, 1), jnp.float32),
        scratch_shapes=[pltpu.SMEM((2,), jnp.float32)],
    )(s8, t8, fh)
    return loss[0, 0]


def kernel(pred, target):
    pred3 = pred.reshape(_N, _C, _HW)
    target3 = target.reshape(_N, _C, _HW)
    return _run(pred3, target3, jnp.asarray(_FH).reshape(_N, 1, _HW))


# speedup vs baseline: 1.0677x; 1.0159x over previous
import jax
import jax.numpy as jnp
import numpy as np
from jax.experimental import pallas as pl
from jax.experimental.pallas import tpu as pltpu

_N, _C, _H, _W = 4, 96, 224, 224
_HW = _H * _W
_RATIO = 2
_MAXN = _HW * _RATIO
_CB = 8
_NCB = _C // _CB


def _rotl(x, d):
    return ((x << np.uint32(d)) | (x >> np.uint32(32 - d))).astype(np.uint32)


def _threefry2x32(k0, k1, x0, x1):
    x0 = x0.astype(np.uint32).copy()
    x1 = x1.astype(np.uint32).copy()
    ks2 = np.uint32(k0 ^ k1 ^ np.uint32(0x1BD11BDA))
    rot = [(13, 15, 26, 6), (17, 29, 16, 24)]
    x0 = (x0 + k0).astype(np.uint32)
    x1 = (x1 + k1).astype(np.uint32)
    ks = [k0, k1, ks2]
    for i in range(5):
        for r in rot[i % 2]:
            x0 = (x0 + x1).astype(np.uint32)
            x1 = _rotl(x1, r) ^ x0
        x0 = (x0 + ks[(i + 1) % 3]).astype(np.uint32)
        x1 = (x1 + ks[(i + 2) % 3] + np.uint32(i + 1)).astype(np.uint32)
    return x0, x1


def _np_fold_in(key, data):
    o0, o1 = _threefry2x32(key[0], key[1], np.array([0], np.uint32),
                           np.array([data], np.uint32))
    return np.array([o0[0], o1[0]], np.uint32)


def _np_random_bits(key, n):
    b1, b2 = _threefry2x32(key[0], key[1], np.zeros(n, np.uint32),
                           np.arange(n, dtype=np.uint32))
    return b1 ^ b2


def _np_split(key):
    b1, b2 = _threefry2x32(key[0], key[1], np.zeros(2, np.uint32),
                           np.array([0, 1], np.uint32))
    return (np.array([b1[0], b2[0]], np.uint32),
            np.array([b1[1], b2[1]], np.uint32))


def _np_randint(key, n, maxval):
    k1, k2 = _np_split(key)
    y = _np_random_bits(k1, n)
    z = _np_random_bits(k2, n)
    s = np.uint32(maxval)
    mult = ((np.uint32(65536) % s) ** 2) % s
    return (((y % s) * mult + (z % s)) % s).astype(np.int64)


def _first_hit_table() -> np.ndarray:
    base = np.array([0, 1234], np.uint32)
    rows = []
    js_rev = np.arange(_MAXN, dtype=np.int32)[::-1]
    for i in range(_N):
        xs = _np_randint(_np_fold_in(base, 2 * i), _MAXN, _W)
        ys = _np_randint(_np_fold_in(base, 2 * i + 1), _MAXN, _H)
        pos = ys * _W + xs
        fh = np.full(_HW, _MAXN, np.int32)
        fh[pos[::-1]] = js_rev
        rows.append(fh)
    return np.stack(rows)


_FH = _first_hit_table()


_TILE = 3584


def _dense_body(pred_ref, target_ref, s8_ref, t8_ref):
    def chunk(c, carry):
        sl = pl.ds(c * 256, 256)
        x = pred_ref[0, :, sl]
        z = target_ref[0, :, sl]
        bce = jnp.maximum(x, 0.0) - x * z + jnp.log1p(jnp.exp(-jnp.abs(x)))
        s = bce[0:8]
        zm = z[0:8]
        for r in range(8, _C, 8):
            s = s + bce[r:r + 8]
            zm = jnp.maximum(zm, z[r:r + 8])
        s8_ref[0, :, sl] = s
        t8_ref[0, :, sl] = zm
        return carry

    jax.lax.fori_loop(0, _TILE // 256, chunk, 0)


def _combine_body(s8_ref, t8_ref, fh_ref, out_ref, acc_ref):
    i = pl.program_id(0)
    tm = jnp.max(t8_ref[0], axis=0)
    num = jnp.sum(tm).astype(jnp.int32) * _RATIO
    s = jnp.sum(s8_ref[0], axis=0)
    m = (tm > 0.0).astype(jnp.float32)
    w = jnp.maximum(m, (fh_ref[0, 0] < num).astype(jnp.float32))
    n_part = jnp.sum(w * s)
    d_part = jnp.sum(w)

    @pl.when(i == 0)
    def _():
        acc_ref[0] = n_part
        acc_ref[1] = d_part

    @pl.when(i != 0)
    def _():
        acc_ref[0] += n_part
        acc_ref[1] += d_part

    @pl.when(i == _N - 1)
    def _():
        out_ref[0, 0] = acc_ref[0] / acc_ref[1]


@jax.jit
def _run(pred3, target3, fh):
    s8, t8 = pl.pallas_call(
        _dense_body,
        grid=(_N, _HW // _TILE),
        in_specs=[
            pl.BlockSpec((1, _C, _TILE), lambda i, t: (i, 0, t)),
            pl.BlockSpec((1, _C, _TILE), lambda i, t: (i, 0, t)),
        ],
        out_specs=[
            pl.BlockSpec((1, _CB, _TILE), lambda i, t: (i, 0, t)),
            pl.BlockSpec((1, _CB, _TILE), lambda i, t: (i, 0, t)),
        ],
        out_shape=[
            jax.ShapeDtypeStruct((_N, _CB, _HW), jnp.float32),
            jax.ShapeDtypeStruct((_N, _CB, _HW), jnp.float32),
        ],
    )(pred3, target3)

    loss = pl.pallas_call(
        _combine_body,
        grid=(_N,),
        in_specs=[
            pl.BlockSpec((1, _CB, _HW), lambda i: (i, 0, 0)),
            pl.BlockSpec((1, _CB, _HW), lambda i: (i, 0, 0)),
            pl.BlockSpec((1, 1, _HW), lambda i: (i, 0, 0)),
        ],
        out_specs=pl.BlockSpec((1, 1), lambda i: (0, 0),
                               memory_space=pltpu.SMEM),
        out_shape=jax.ShapeDtypeStruct((1, 1), jnp.float32),
        scratch_shapes=[pltpu.SMEM((2,), jnp.float32)],
    )(s8, t8, fh)
    return loss[0, 0]


def kernel(pred, target):
    pred3 = pred.reshape(_N, _C, _HW)
    target3 = target.reshape(_N, _C, _HW)
    return _run(pred3, target3, jnp.asarray(_FH).reshape(_N, 1, _HW))


# speedup vs baseline: 1.0887x; 1.0196x over previous
import jax
import jax.numpy as jnp
import numpy as np
from jax.experimental import pallas as pl
from jax.experimental.pallas import tpu as pltpu

_N, _C, _H, _W = 4, 96, 224, 224
_HW = _H * _W
_RATIO = 2
_MAXN = _HW * _RATIO
_CB = 8
_NCB = _C // _CB


def _rotl(x, d):
    return ((x << np.uint32(d)) | (x >> np.uint32(32 - d))).astype(np.uint32)


def _threefry2x32(k0, k1, x0, x1):
    x0 = x0.astype(np.uint32).copy()
    x1 = x1.astype(np.uint32).copy()
    ks2 = np.uint32(k0 ^ k1 ^ np.uint32(0x1BD11BDA))
    rot = [(13, 15, 26, 6), (17, 29, 16, 24)]
    x0 = (x0 + k0).astype(np.uint32)
    x1 = (x1 + k1).astype(np.uint32)
    ks = [k0, k1, ks2]
    for i in range(5):
        for r in rot[i % 2]:
            x0 = (x0 + x1).astype(np.uint32)
            x1 = _rotl(x1, r) ^ x0
        x0 = (x0 + ks[(i + 1) % 3]).astype(np.uint32)
        x1 = (x1 + ks[(i + 2) % 3] + np.uint32(i + 1)).astype(np.uint32)
    return x0, x1


def _np_fold_in(key, data):
    o0, o1 = _threefry2x32(key[0], key[1], np.array([0], np.uint32),
                           np.array([data], np.uint32))
    return np.array([o0[0], o1[0]], np.uint32)


def _np_random_bits(key, n):
    b1, b2 = _threefry2x32(key[0], key[1], np.zeros(n, np.uint32),
                           np.arange(n, dtype=np.uint32))
    return b1 ^ b2


def _np_split(key):
    b1, b2 = _threefry2x32(key[0], key[1], np.zeros(2, np.uint32),
                           np.array([0, 1], np.uint32))
    return (np.array([b1[0], b2[0]], np.uint32),
            np.array([b1[1], b2[1]], np.uint32))


def _np_randint(key, n, maxval):
    k1, k2 = _np_split(key)
    y = _np_random_bits(k1, n)
    z = _np_random_bits(k2, n)
    s = np.uint32(maxval)
    mult = ((np.uint32(65536) % s) ** 2) % s
    return (((y % s) * mult + (z % s)) % s).astype(np.int64)


def _first_hit_table() -> np.ndarray:
    base = np.array([0, 1234], np.uint32)
    rows = []
    js_rev = np.arange(_MAXN, dtype=np.int32)[::-1]
    for i in range(_N):
        xs = _np_randint(_np_fold_in(base, 2 * i), _MAXN, _W)
        ys = _np_randint(_np_fold_in(base, 2 * i + 1), _MAXN, _H)
        pos = ys * _W + xs
        fh = np.full(_HW, _MAXN, np.int32)
        fh[pos[::-1]] = js_rev
        rows.append(fh)
    return np.stack(rows)


_FH = _first_hit_table()


_TILE = 3584


def _dense_body(pred_ref, target_ref, s8_ref, t8_ref):
    def chunk(c, carry):
        sl = pl.ds(c * 256, 256)
        x = pred_ref[0, :, sl]
        z = target_ref[0, :, sl]
        e = jnp.exp2(jnp.abs(x) * (-1.4426950408889634))
        l1pe = jnp.log2(1.0 + e) * 0.6931471805599453
        bce = jnp.maximum(x, 0.0) - x * z + l1pe
        s = bce[0:8]
        zm = z[0:8]
        for r in range(8, _C, 8):
            s = s + bce[r:r + 8]
            zm = jnp.maximum(zm, z[r:r + 8])
        s8_ref[0, :, sl] = s
        t8_ref[0, :, sl] = zm
        return carry

    jax.lax.fori_loop(0, _TILE // 256, chunk, 0)


def _combine_body(s8_ref, t8_ref, fh_ref, out_ref, acc_ref):
    i = pl.program_id(0)
    tm = jnp.max(t8_ref[0], axis=0)
    num = jnp.sum(tm).astype(jnp.int32) * _RATIO
    s = jnp.sum(s8_ref[0], axis=0)
    m = (tm > 0.0).astype(jnp.float32)
    w = jnp.maximum(m, (fh_ref[0, 0] < num).astype(jnp.float32))
    n_part = jnp.sum(w * s)
    d_part = jnp.sum(w)

    @pl.when(i == 0)
    def _():
        acc_ref[0] = n_part
        acc_ref[1] = d_part

    @pl.when(i != 0)
    def _():
        acc_ref[0] += n_part
        acc_ref[1] += d_part

    @pl.when(i == _N - 1)
    def _():
        out_ref[0, 0] = acc_ref[0] / acc_ref[1]


@jax.jit
def _run(pred3, target3, fh):
    s8, t8 = pl.pallas_call(
        _dense_body,
        grid=(_N, _HW // _TILE),
        in_specs=[
            pl.BlockSpec((1, _C, _TILE), lambda i, t: (i, 0, t)),
            pl.BlockSpec((1, _C, _TILE), lambda i, t: (i, 0, t)),
        ],
        out_specs=[
            pl.BlockSpec((1, _CB, _TILE), lambda i, t: (i, 0, t)),
            pl.BlockSpec((1, _CB, _TILE), lambda i, t: (i, 0, t)),
        ],
        out_shape=[
            jax.ShapeDtypeStruct((_N, _CB, _HW), jnp.float32),
            jax.ShapeDtypeStruct((_N, _CB, _HW), jnp.float32),
        ],
    )(pred3, target3)

    loss = pl.pallas_call(
        _combine_body,
        grid=(_N,),
        in_specs=[
            pl.BlockSpec((1, _CB, _HW), lambda i: (i, 0, 0)),
            pl.BlockSpec((1, _CB, _HW), lambda i: (i, 0, 0)),
            pl.BlockSpec((1, 1, _HW), lambda i: (i, 0, 0)),
        ],
        out_specs=pl.BlockSpec((1, 1), lambda i: (0, 0),
                               memory_space=pltpu.SMEM),
        out_shape=jax.ShapeDtypeStruct((1, 1), jnp.float32),
        scratch_shapes=[pltpu.SMEM((2,), jnp.float32)],
    )(s8, t8, fh)
    return loss[0, 0]


def kernel(pred, target):
    pred3 = pred.reshape(_N, _C, _HW)
    target3 = target.reshape(_N, _C, _HW)
    return _run(pred3, target3, jnp.asarray(_FH).reshape(_N, 1, _HW))


# speedup vs baseline: 1.0964x; 1.0071x over previous
import jax
import jax.numpy as jnp
import numpy as np
from jax.experimental import pallas as pl
from jax.experimental.pallas import tpu as pltpu

_N, _C, _H, _W = 4, 96, 224, 224
_HW = _H * _W
_RATIO = 2
_MAXN = _HW * _RATIO
_CB = 8
_NCB = _C // _CB


def _rotl(x, d):
    return ((x << np.uint32(d)) | (x >> np.uint32(32 - d))).astype(np.uint32)


def _threefry2x32(k0, k1, x0, x1):
    x0 = x0.astype(np.uint32).copy()
    x1 = x1.astype(np.uint32).copy()
    ks2 = np.uint32(k0 ^ k1 ^ np.uint32(0x1BD11BDA))
    rot = [(13, 15, 26, 6), (17, 29, 16, 24)]
    x0 = (x0 + k0).astype(np.uint32)
    x1 = (x1 + k1).astype(np.uint32)
    ks = [k0, k1, ks2]
    for i in range(5):
        for r in rot[i % 2]:
            x0 = (x0 + x1).astype(np.uint32)
            x1 = _rotl(x1, r) ^ x0
        x0 = (x0 + ks[(i + 1) % 3]).astype(np.uint32)
        x1 = (x1 + ks[(i + 2) % 3] + np.uint32(i + 1)).astype(np.uint32)
    return x0, x1


def _np_fold_in(key, data):
    o0, o1 = _threefry2x32(key[0], key[1], np.array([0], np.uint32),
                           np.array([data], np.uint32))
    return np.array([o0[0], o1[0]], np.uint32)


def _np_random_bits(key, n):
    b1, b2 = _threefry2x32(key[0], key[1], np.zeros(n, np.uint32),
                           np.arange(n, dtype=np.uint32))
    return b1 ^ b2


def _np_split(key):
    b1, b2 = _threefry2x32(key[0], key[1], np.zeros(2, np.uint32),
                           np.array([0, 1], np.uint32))
    return (np.array([b1[0], b2[0]], np.uint32),
            np.array([b1[1], b2[1]], np.uint32))


def _np_randint(key, n, maxval):
    k1, k2 = _np_split(key)
    y = _np_random_bits(k1, n)
    z = _np_random_bits(k2, n)
    s = np.uint32(maxval)
    mult = ((np.uint32(65536) % s) ** 2) % s
    return (((y % s) * mult + (z % s)) % s).astype(np.int64)


def _first_hit_table() -> np.ndarray:
    base = np.array([0, 1234], np.uint32)
    rows = []
    js_rev = np.arange(_MAXN, dtype=np.int32)[::-1]
    for i in range(_N):
        xs = _np_randint(_np_fold_in(base, 2 * i), _MAXN, _W)
        ys = _np_randint(_np_fold_in(base, 2 * i + 1), _MAXN, _H)
        pos = ys * _W + xs
        fh = np.full(_HW, _MAXN, np.int32)
        fh[pos[::-1]] = js_rev
        rows.append(fh)
    return np.stack(rows)


_FH = _first_hit_table()


_TILE = 3584


def _dense_body(pred_ref, target_ref, s8_ref, t8_ref):
    def chunk(c, carry):
        sl = pl.ds(c * 256, 256)
        x = pred_ref[0, :, sl]
        z = target_ref[0, :, sl]
        e = jnp.exp2(jnp.abs(x) * (-1.4426950408889634))
        l1pe = jnp.log2(1.0 + e) * 0.6931471805599453
        bce = jnp.maximum(x, 0.0) - x * z + l1pe
        s = bce[0:8]
        zm = z[0:8]
        for r in range(8, _C, 8):
            s = s + bce[r:r + 8]
            zm = jnp.maximum(zm, z[r:r + 8])
        s8_ref[0, :, sl] = s
        t8_ref[0, :, sl] = zm
        return carry

    jax.lax.fori_loop(0, _TILE // 256, chunk, 0, unroll=2)


def _combine_body(s8_ref, t8_ref, fh_ref, out_ref, acc_ref):
    i = pl.program_id(0)
    tm = jnp.max(t8_ref[0], axis=0)
    num = jnp.sum(tm).astype(jnp.int32) * _RATIO
    s = jnp.sum(s8_ref[0], axis=0)
    m = (tm > 0.0).astype(jnp.float32)
    w = jnp.maximum(m, (fh_ref[0, 0] < num).astype(jnp.float32))
    n_part = jnp.sum(w * s)
    d_part = jnp.sum(w)

    @pl.when(i == 0)
    def _():
        acc_ref[0] = n_part
        acc_ref[1] = d_part

    @pl.when(i != 0)
    def _():
        acc_ref[0] += n_part
        acc_ref[1] += d_part

    @pl.when(i == _N - 1)
    def _():
        out_ref[0, 0] = acc_ref[0] / acc_ref[1]


@jax.jit
def _run(pred3, target3, fh):
    s8, t8 = pl.pallas_call(
        _dense_body,
        grid=(_N, _HW // _TILE),
        in_specs=[
            pl.BlockSpec((1, _C, _TILE), lambda i, t: (i, 0, t)),
            pl.BlockSpec((1, _C, _TILE), lambda i, t: (i, 0, t)),
        ],
        out_specs=[
            pl.BlockSpec((1, _CB, _TILE), lambda i, t: (i, 0, t)),
            pl.BlockSpec((1, _CB, _TILE), lambda i, t: (i, 0, t)),
        ],
        out_shape=[
            jax.ShapeDtypeStruct((_N, _CB, _HW), jnp.float32),
            jax.ShapeDtypeStruct((_N, _CB, _HW), jnp.float32),
        ],
    )(pred3, target3)

    loss = pl.pallas_call(
        _combine_body,
        grid=(_N,),
        in_specs=[
            pl.BlockSpec((1, _CB, _HW), lambda i: (i, 0, 0)),
            pl.BlockSpec((1, _CB, _HW), lambda i: (i, 0, 0)),
            pl.BlockSpec((1, 1, _HW), lambda i: (i, 0, 0)),
        ],
        out_specs=pl.BlockSpec((1, 1), lambda i: (0, 0),
                               memory_space=pltpu.SMEM),
        out_shape=jax.ShapeDtypeStruct((1, 1), jnp.float32),
        scratch_shapes=[pltpu.SMEM((2,), jnp.float32)],
    )(s8, t8, fh)
    return loss[0, 0]


def kernel(pred, target):
    pred3 = pred.reshape(_N, _C, _HW)
    target3 = target.reshape(_N, _C, _HW)
    return _run(pred3, target3, jnp.asarray(_FH).reshape(_N, 1, _HW))
